# baseline (speedup 1.0000x reference)
import functools

import jax
import jax.numpy as jnp
from jax import lax
from jax.experimental import pallas as pl
from jax.experimental.pallas import tpu as pltpu
from jax.experimental.pallas import tpu_sc as plsc

MAZE = 32
P = MAZE * MAZE
D = 64
V = 4
PP = P // 2
TBL = V * V * PP
NC, NS, L = 2, 16, 16
NW = NC * NS
CHP = 64
CPM = PP // CHP
NCB = 4


def _sc_encode(me, mo, cellcat, pos128, batch):
    nb = batch // NW
    rows_per_sub = TBL // NS

    mesh = plsc.VectorSubcoreMesh(core_axis_name="c", subcore_axis_name="s")

    @functools.partial(
        pl.kernel,
        out_type=jax.ShapeDtypeStruct((batch, PP, 2 * D), jnp.float32),
        mesh=mesh,
        compiler_params=pltpu.CompilerParams(use_tc_tiling_on_sc=False),
        scratch_types=[
            pltpu.VMEM_SHARED((TBL, 2 * D), jnp.float32),
            pltpu.VMEM((2 * D,), jnp.float32),
        ] + [pltpu.VMEM((PP,), jnp.float32) for _ in range(4)]
          + [pltpu.VMEM((CHP,), jnp.int32) for _ in range(NCB)]
          + [pltpu.VMEM((CHP, 2 * D), jnp.float32) for _ in range(NCB)]
          + [pltpu.SemaphoreType.DMA for _ in range(2 + 2 * NCB)],
    )
    def k(me_hbm, mo_hbm, cell_hbm, pos_hbm, out_hbm, tbl_sh, ccbuf, *ring):
        mebufs = ring[0:2]
        mobufs = ring[2:4]
        ibufs = ring[4:4 + NCB]
        cbufs = ring[4 + NCB:4 + 2 * NCB]
        msems = ring[4 + 2 * NCB:6 + 2 * NCB]
        gsems = ring[6 + 2 * NCB:6 + 3 * NCB]
        osems = ring[6 + 3 * NCB:6 + 4 * NCB]
        cid = lax.axis_index("c")
        sid = lax.axis_index("s")
        wid = sid * NC + cid
        b0 = wid * nb
        lanes = lax.iota(jnp.int32, L)

        pltpu.sync_copy(cell_hbm.at[sid], ccbuf)
        ccs = [ccbuf[pl.ds(j * L, L)] for j in range(2 * D // L)]
        row0 = sid * rows_per_sub
        for kk in range(rows_per_sub // CHP):
            bb = cbufs[kk % 2]
            pltpu.sync_copy(pos_hbm.at[pl.ds(kk * CHP, CHP)], bb)

            def add_row(r, _, bb=bb):
                for j in range(2 * D // L):
                    bb[r, pl.ds(j * L, L)] += ccs[j]
                return _

            lax.fori_loop(0, CHP, add_row, 0)
            pltpu.sync_copy(bb, tbl_sh.at[pl.ds(row0 + kk * CHP, CHP)])
        plsc.subcore_barrier()

        def issue_maze(m, mb):
            pltpu.async_copy(me_hbm.at[pl.ds((b0 + m) * PP, PP)], mebufs[mb],
                             msems[mb])
            pltpu.async_copy(mo_hbm.at[pl.ds((b0 + m) * PP, PP)], mobufs[mb],
                             msems[mb])

        def wait_maze(m, mb):
            pltpu.make_async_copy(me_hbm.at[pl.ds((b0 + m) * PP, PP)],
                                  mebufs[mb], msems[mb]).wait()
            pltpu.make_async_copy(mo_hbm.at[pl.ds((b0 + m) * PP, PP)],
                                  mobufs[mb], msems[mb]).wait()

        def build_and_gather(q, s, mb):
            for jj in range(CHP // L):
                e = mebufs[mb][pl.ds(q * CHP + jj * L, L)].astype(jnp.int32)
                o = mobufs[mb][pl.ds(q * CHP + jj * L, L)].astype(jnp.int32)
                pid = q * CHP + jj * L
                ibufs[s][pl.ds(jj * L, L)] = (e * V + o) * PP + pid + lanes
            pltpu.async_copy(tbl_sh.at[ibufs[s]], cbufs[s], gsems[s])

        def wait_gather(s):
            pltpu.make_async_copy(tbl_sh.at[ibufs[s]], cbufs[s],
                                  gsems[s]).wait()

        def _out_slice(m, q):
            return out_hbm.at[b0 + m, pl.ds(q * CHP, CHP)]

        def issue_out(m, q, s):
            pltpu.async_copy(cbufs[s], _out_slice(m, q), osems[s])

        def wait_out(m, q, s):
            pltpu.make_async_copy(cbufs[s], _out_slice(m, q), osems[s]).wait()

        def chunk_step(m, q, mb, first=False, last=False):
            s = q % NCB
            s2 = (q + 2) % NCB
            wait_gather(s)
            issue_out(m, q, s)
            if q == 6 and not last:
                wait_maze(m + 1, 1 - mb)
            if not (first and q < 2):
                pq, pm = (q - 2, m) if q >= 2 else (q + CPM - 2, m - 1)
                wait_out(pm, pq, s2)
            if not (last and q >= 6):
                if q < 6:
                    build_and_gather(q + 2, s2, mb)
                else:
                    build_and_gather(q - 6, s2, 1 - mb)

        issue_maze(0, 0)
        issue_maze(1, 1)
        wait_maze(0, 0)
        build_and_gather(0, 0, 0)
        build_and_gather(1, 1, 0)

        def maze_pair(mi, first=False, last=False):
            for mb in range(2):
                m = mi * 2 + mb
                for q in range(CPM):
                    if q == 6 and not (last and mb == 1):
                        if not (last and mb == 0):
                            issue_maze(m + 2, mb)
                    chunk_step(m, q, mb,
                               first=(first and mb == 0),
                               last=(last and mb == 1))

        maze_pair(0, first=True)

        def group(mi, _):
            maze_pair(mi)
            return _

        lax.fori_loop(1, nb // 2 - 1, group, 0)
        maze_pair(nb // 2 - 1, last=True)

        for q in range(CPM - 2, CPM):
            wait_out(nb - 1, q, q % NCB)

    return k(me, mo, cellcat, pos128)


def kernel(maze_grid, cell_table, pos_table):
    batch, h, w = maze_grid.shape
    mz = maze_grid.reshape(-1, 2).astype(jnp.float32)
    me = mz[:, 0]
    mo = mz[:, 1]
    cellcat = jnp.concatenate(
        [jnp.repeat(cell_table, V, axis=0),
         jnp.tile(cell_table, (V, 1))], axis=1)
    pos128 = pos_table.reshape(PP, 2 * D)
    out = _sc_encode(me, mo, cellcat, pos128, batch)
    return out.reshape(batch, h * w, D)
